# baseline (speedup 1.0000x reference)
import functools

import jax
import jax.numpy as jnp
from jax import lax
from jax.experimental import pallas as pl
from jax.experimental.pallas import tpu as pltpu
from jax.experimental.pallas import tpu_sc as plsc

VOCAB_ = 100000
EMB_ = 64
HID_ = 64
B_ = 1024
L_ = 50

_NC = 2
_NS = 16
_NW = _NC * _NS
_BW = B_ // _NW


def _sc_gather_body(idx_hbm, table_hbm, out_hbm, blk_v, idx_v, rows_v, sem):
    wid = lax.axis_index("s") * _NC + lax.axis_index("c")
    b0 = wid * _BW
    pltpu.sync_copy(idx_hbm.at[pl.ds(b0, _BW)], blk_v)

    def tr(g, carry):
        q = lax.iota(jnp.int32, 16) + g * 16
        vals = plsc.load_gather(blk_v, [q & (_BW - 1), q >> 5])
        idx_v[pl.ds(pl.multiple_of(g * 16, 16), 16)] = vals
        return carry

    lax.fori_loop(0, _BW * L_ // 16, tr, 0)

    def gth(j, carry):
        pltpu.async_copy(
            table_hbm.at[idx_v.at[pl.ds(pl.multiple_of(j * _BW, _BW), _BW)]],
            rows_v.at[j], sem)
        return carry

    lax.fori_loop(0, L_, gth, 0)
    pltpu.make_async_copy(out_hbm.at[:, pl.ds(b0, _BW)], rows_v, sem).wait()
    pltpu.sync_copy(rows_v, out_hbm.at[:, pl.ds(b0, _BW)])


@functools.lru_cache(maxsize=None)
def _sc_gather():
    return pl.kernel(
        _sc_gather_body,
        out_type=jax.ShapeDtypeStruct((L_, B_, EMB_), jnp.float32),
        mesh=plsc.VectorSubcoreMesh(core_axis_name="c", subcore_axis_name="s"),
        scratch_types=[
            pltpu.VMEM((_BW, EMB_), jnp.int32),
            pltpu.VMEM((_BW * L_,), jnp.int32),
            pltpu.VMEM((L_, _BW, EMB_), jnp.float32),
            pltpu.SemaphoreType.DMA,
        ],
        compiler_params=pltpu.CompilerParams(
            use_tc_tiling_on_sc=False, needs_layout_passes=False),
    )


_BP = B_ // 2
_W2 = 2 * HID_


def _rnn_body(x_ref, wih_ref, whh_ref, b_ref, out_ref):
    x_all = x_ref[...].reshape(L_ * _BP, _W2)
    a = jnp.dot(x_all, wih_ref[...], preferred_element_type=jnp.float32)
    out_ref[...] = (a + b_ref[...]).reshape(L_, _BP, _W2)

    def step(t, h):
        hn = jnp.tanh(
            out_ref[t]
            + jnp.dot(h, whh_ref[...], preferred_element_type=jnp.float32)
        )
        out_ref[t] = hn
        return hn

    lax.fori_loop(0, L_, step, jnp.zeros((_BP, _W2), jnp.float32))


_rnn = pl.pallas_call(
    _rnn_body,
    out_shape=jax.ShapeDtypeStruct((L_, _BP, _W2), jnp.float32),
)


def _sc_tr_body(ys_hbm, out_hbm, h_hbm, buf_v, sem):
    wid = lax.axis_index("s") * _NC + lax.axis_index("c")
    b0 = wid * _BW
    for bl in range(_BW):
        pltpu.async_copy(ys_hbm.at[:, b0 + bl, :], buf_v.at[bl], sem)
    pltpu.make_async_copy(out_hbm.at[pl.ds(b0, _BW)], buf_v, sem).wait()
    pltpu.sync_copy(buf_v.at[:, L_ - 1, :], h_hbm.at[0, pl.ds(b0, _BW)])
    for bl in range(_BW):
        pltpu.async_copy(buf_v.at[bl], out_hbm.at[b0 + bl], sem)
    pltpu.make_async_copy(out_hbm.at[pl.ds(b0, _BW)], buf_v, sem).wait()


@functools.lru_cache(maxsize=None)
def _sc_tr():
    return pl.kernel(
        _sc_tr_body,
        out_type=(
            jax.ShapeDtypeStruct((B_, L_, HID_), jnp.float32),
            jax.ShapeDtypeStruct((1, B_, HID_), jnp.float32),
        ),
        mesh=plsc.VectorSubcoreMesh(core_axis_name="c", subcore_axis_name="s"),
        scratch_types=[
            pltpu.VMEM((_BW, L_, HID_), jnp.float32),
            pltpu.SemaphoreType.DMA,
        ],
        compiler_params=pltpu.CompilerParams(
            use_tc_tiling_on_sc=False, needs_layout_passes=False),
    )


def _blockdiag2(w):
    z = jnp.zeros((HID_, HID_), w.dtype)
    return jnp.block([[w, z], [z, w]])


def kernel(sentences, emb_table, W_ih, W_hh, b_ih, b_hh):
    idx = jnp.pad(sentences.astype(jnp.int32), ((0, 0), (0, EMB_ - L_)))
    x = _sc_gather()(idx, emb_table)
    x2 = x.reshape(L_, _BP, _W2)
    bias1 = b_ih + b_hh
    bias2 = jnp.concatenate([bias1, bias1]).reshape(1, _W2)
    ys2 = _rnn(x2, _blockdiag2(W_ih.T), _blockdiag2(W_hh.T), bias2)
    ys = ys2.reshape(L_, B_, HID_)
    final_output, h = _sc_tr()(ys)
    return final_output, h


# speedup vs baseline: 1.1214x; 1.1214x over previous
import functools

import jax
import jax.numpy as jnp
from jax import lax
from jax.experimental import pallas as pl
from jax.experimental.pallas import tpu as pltpu
from jax.experimental.pallas import tpu_sc as plsc

VOCAB_ = 100000
EMB_ = 64
HID_ = 64
B_ = 1024
L_ = 50

_NC = 2
_NS = 16
_NW = _NC * _NS
_BW = B_ // _NW


def _sc_gather_body(idx_hbm, table_hbm, out_hbm, blk_v, idx_v, rows_v, sem):
    wid = lax.axis_index("s") * _NC + lax.axis_index("c")
    b0 = wid * _BW
    pltpu.sync_copy(idx_hbm.at[pl.ds(b0, _BW)], blk_v)

    def tr(g, carry):
        q = lax.iota(jnp.int32, 16) + g * 16
        vals = plsc.load_gather(blk_v, [q & (_BW - 1), q >> 5])
        idx_v[pl.ds(pl.multiple_of(g * 16, 16), 16)] = vals
        return carry

    lax.fori_loop(0, _BW * L_ // 16, tr, 0)

    def gth(j, carry):
        pltpu.async_copy(
            table_hbm.at[idx_v.at[pl.ds(pl.multiple_of(j * _BW, _BW), _BW)]],
            rows_v.at[j], sem)
        return carry

    lax.fori_loop(0, L_, gth, 0)
    pltpu.make_async_copy(out_hbm.at[:, pl.ds(b0, _BW)], rows_v, sem).wait()
    pltpu.sync_copy(rows_v, out_hbm.at[:, pl.ds(b0, _BW)])


@functools.lru_cache(maxsize=None)
def _sc_gather():
    return pl.kernel(
        _sc_gather_body,
        out_type=jax.ShapeDtypeStruct((L_, B_, EMB_), jnp.float32),
        mesh=plsc.VectorSubcoreMesh(core_axis_name="c", subcore_axis_name="s"),
        scratch_types=[
            pltpu.VMEM((_BW, EMB_), jnp.int32),
            pltpu.VMEM((_BW * L_,), jnp.int32),
            pltpu.VMEM((L_, _BW, EMB_), jnp.float32),
            pltpu.SemaphoreType.DMA,
        ],
        compiler_params=pltpu.CompilerParams(
            use_tc_tiling_on_sc=False, needs_layout_passes=False),
    )


_BP = B_ // 2
_W2 = 2 * HID_


_CH = 10


def _rnn_body(x_ref, wih_ref, whh_ref, b_ref, out_ref, h_ref):
    g = pl.program_id(0)

    @pl.when(g == 0)
    def _():
        h_ref[...] = jnp.zeros_like(h_ref)

    x_all = x_ref[...].reshape(_CH * _BP, _W2)
    a = jnp.dot(x_all, wih_ref[...], preferred_element_type=jnp.float32)
    out_ref[...] = (a + b_ref[...]).reshape(_CH, _BP, _W2)

    def step(c, h):
        hn = jnp.tanh(
            out_ref[c]
            + jnp.dot(h, whh_ref[...], preferred_element_type=jnp.float32)
        )
        out_ref[c] = hn
        return hn

    h_ref[...] = lax.fori_loop(0, _CH, step, h_ref[...])


_rnn = pl.pallas_call(
    _rnn_body,
    grid=(L_ // _CH,),
    in_specs=[
        pl.BlockSpec((_CH, _BP, _W2), lambda g: (g, 0, 0)),
        pl.BlockSpec((_W2, _W2), lambda g: (0, 0)),
        pl.BlockSpec((_W2, _W2), lambda g: (0, 0)),
        pl.BlockSpec((1, _W2), lambda g: (0, 0)),
    ],
    out_specs=pl.BlockSpec((_CH, _BP, _W2), lambda g: (g, 0, 0)),
    out_shape=jax.ShapeDtypeStruct((L_, _BP, _W2), jnp.float32),
    scratch_shapes=[pltpu.VMEM((_BP, _W2), jnp.float32)],
)


def _sc_tr_body(ys_hbm, out_hbm, h_hbm, buf_v, sem):
    wid = lax.axis_index("s") * _NC + lax.axis_index("c")
    b0 = wid * _BW
    for bl in range(_BW):
        pltpu.async_copy(ys_hbm.at[:, b0 + bl, :], buf_v.at[bl], sem)
    pltpu.make_async_copy(out_hbm.at[pl.ds(b0, _BW)], buf_v, sem).wait()
    pltpu.sync_copy(buf_v.at[:, L_ - 1, :], h_hbm.at[0, pl.ds(b0, _BW)])
    for bl in range(_BW):
        pltpu.async_copy(buf_v.at[bl], out_hbm.at[b0 + bl], sem)
    pltpu.make_async_copy(out_hbm.at[pl.ds(b0, _BW)], buf_v, sem).wait()


@functools.lru_cache(maxsize=None)
def _sc_tr():
    return pl.kernel(
        _sc_tr_body,
        out_type=(
            jax.ShapeDtypeStruct((B_, L_, HID_), jnp.float32),
            jax.ShapeDtypeStruct((1, B_, HID_), jnp.float32),
        ),
        mesh=plsc.VectorSubcoreMesh(core_axis_name="c", subcore_axis_name="s"),
        scratch_types=[
            pltpu.VMEM((_BW, L_, HID_), jnp.float32),
            pltpu.SemaphoreType.DMA,
        ],
        compiler_params=pltpu.CompilerParams(
            use_tc_tiling_on_sc=False, needs_layout_passes=False),
    )


def _blockdiag2(w):
    z = jnp.zeros((HID_, HID_), w.dtype)
    return jnp.block([[w, z], [z, w]])


def kernel(sentences, emb_table, W_ih, W_hh, b_ih, b_hh):
    idx = jnp.pad(sentences.astype(jnp.int32), ((0, 0), (0, EMB_ - L_)))
    x = _sc_gather()(idx, emb_table)
    x2 = x.reshape(L_, _BP, _W2)
    bias1 = b_ih + b_hh
    bias2 = jnp.concatenate([bias1, bias1]).reshape(1, _W2)
    ys2 = _rnn(x2, _blockdiag2(W_ih.T), _blockdiag2(W_hh.T), bias2)
    ys = ys2.reshape(L_, B_, HID_)
    final_output = jnp.swapaxes(ys, 0, 1)
    h = ys[L_ - 1][None, :, :]
    return final_output, h
